# Initial kernel scaffold; baseline (speedup 1.0000x reference)
import jax
import jax.numpy as jnp
from jax.experimental import pallas as pl


def kernel(embryo_type, table, W, b, gamma, beta):
    raise NotImplementedError("write your pallas kernel here")


# speedup vs baseline: 6.7564x; 6.7564x over previous
import functools
import math

import jax
import jax.numpy as jnp
from jax import lax
from jax.experimental import pallas as pl
from jax.experimental.pallas import tpu as pltpu
from jax.experimental.pallas import tpu_sc as plsc

NUM_EMB = 100000
INNER = 16
EMB = 96
B = 16384
L = 200


_ROWS_PER_BLOCK = 4000
EMB_PAD = 128


def _transform_body(table_ref, w_ref, b_ref, gamma_ref, beta_ref, out_ref):
    x = jnp.dot(table_ref[...], w_ref[...], preferred_element_type=jnp.float32)
    x = x + b_ref[...]
    mean = jnp.sum(x, axis=-1, keepdims=True) * (1.0 / EMB)
    mask = lax.broadcasted_iota(jnp.int32, x.shape, 1) < EMB
    xc = jnp.where(mask, x - mean, 0.0)
    var = jnp.sum(xc * xc, axis=-1, keepdims=True) * (1.0 / EMB)
    y = xc * lax.rsqrt(var + 1e-5)
    y = y * gamma_ref[...] + beta_ref[...]
    out_ref[...] = y * 0.5 * (1.0 + lax.erf(y * (1.0 / math.sqrt(2.0))))


def _transform_table(table, W, b2, gamma2, beta2):
    grid = (NUM_EMB // _ROWS_PER_BLOCK,)
    return pl.pallas_call(
        _transform_body,
        grid=grid,
        in_specs=[
            pl.BlockSpec((_ROWS_PER_BLOCK, INNER), lambda i: (i, 0)),
            pl.BlockSpec((INNER, EMB_PAD), lambda i: (0, 0)),
            pl.BlockSpec((1, EMB_PAD), lambda i: (0, 0)),
            pl.BlockSpec((1, EMB_PAD), lambda i: (0, 0)),
            pl.BlockSpec((1, EMB_PAD), lambda i: (0, 0)),
        ],
        out_specs=pl.BlockSpec((_ROWS_PER_BLOCK, EMB_PAD), lambda i: (i, 0)),
        out_shape=jax.ShapeDtypeStruct((NUM_EMB, EMB_PAD), jnp.float32),
    )(table, W, b2, gamma2, beta2)


N = B * L
_NC, _NS = 2, 16
_NW = _NC * _NS
_PER_W = N // _NW
_CHUNK = 800
_N_CHUNKS = _PER_W // _CHUNK


@functools.cache
def _make_gather_kernel():
    @functools.partial(
        pl.kernel,
        mesh=plsc.VectorSubcoreMesh(core_axis_name="c", subcore_axis_name="s"),
        out_type=jax.ShapeDtypeStruct((N, EMB_PAD), jnp.float32),
        scratch_types=[
            pltpu.VMEM((_CHUNK,), jnp.int32),
            pltpu.VMEM((_CHUNK, EMB_PAD), jnp.float32),
            pltpu.SemaphoreType.DMA,
        ],
    )
    def _gather_kernel(table_hbm, idx_hbm, out_hbm, idx_v, rows_v, sem):
        wid = lax.axis_index("s") * _NC + lax.axis_index("c")
        base = wid * _PER_W

        def body(i, carry):
            off = base + i * _CHUNK
            pltpu.sync_copy(idx_hbm.at[pl.ds(off, _CHUNK)], idx_v)
            pltpu.async_copy(table_hbm.at[idx_v], rows_v, sem).wait()
            pltpu.sync_copy(rows_v, out_hbm.at[pl.ds(off, _CHUNK)])
            return carry

        lax.fori_loop(0, _N_CHUNKS, body, 0)

    return _gather_kernel


def kernel(embryo_type, table, W, b, gamma, beta):
    pad = EMB_PAD - EMB
    table2 = _transform_table(
        table,
        jnp.pad(W, ((0, 0), (0, pad))),
        jnp.pad(b.reshape(1, EMB), ((0, 0), (0, pad))),
        jnp.pad(gamma.reshape(1, EMB), ((0, 0), (0, pad))),
        jnp.pad(beta.reshape(1, EMB), ((0, 0), (0, pad))),
    )
    idx = embryo_type.reshape(N).astype(jnp.int32)
    out = _make_gather_kernel()(table2, idx)
    return out[:, :EMB].reshape(B, L, EMB)
